# baseline (speedup 1.0000x reference)
import jax
import jax.numpy as jnp
from jax.experimental import pallas as pl
from jax.experimental.pallas import tpu as pltpu

B = 2
N = 100000
FEAT = 64
GRID = 512
TE = 4
XMIN = -51.2
VS = 0.2

GG = GRID * GRID
HALF = GG // 2
CHUNK = 2048
NCH = (N + CHUNK - 1) // CHUNK
NPAD = NCH * CHUNK
NP4 = 2 * B
CNT_ROWS = GG // 128
NQ = 2
QH = HALF // NQ
QC = CNT_ROWS // NQ


def _accum_body(seg_ref, pc_ref, wf_ref, bf_ref, sums_ref, cnt_ref,
                acc_ref, cnta_ref, feats_ref, sem1, sem2):
    p = pl.program_id(0)
    q = pl.program_id(1)
    ch = pl.program_id(2)

    @pl.when(ch == 0)
    def _zero():
        acc_ref[...] = jnp.zeros(acc_ref.shape, jnp.float32)
        cnta_ref[...] = jnp.zeros(cnta_ref.shape, jnp.float32)

    feats_ref[...] = jax.nn.relu(pc_ref[0] @ wf_ref[...] + bf_ref[...])

    lane = jax.lax.broadcasted_iota(jnp.int32, (1, 128), 1)
    zero64 = jnp.zeros((1, FEAT), jnp.float32)
    nvalid = jnp.minimum(CHUNK, N - ch * CHUNK)
    seg_lo = q * (GG // NQ)
    seg_hi = seg_lo + GG // NQ

    def body(i, carry):
        s = seg_ref[0, 0, i]

        @pl.when(jnp.logical_and(s >= seg_lo, s < seg_hi))
        def _upd():
            f = feats_ref[pl.ds(i, 1), :]
            placed = jnp.where(s % 2 == 0,
                               jnp.concatenate([f, zero64], axis=1),
                               jnp.concatenate([zero64, f], axis=1))
            r = s // 2 - q * QH
            acc_ref[pl.ds(r, 1), :] = acc_ref[pl.ds(r, 1), :] + placed
            onehot = (lane == s % 128).astype(jnp.float32)
            cr = s // 128 - q * QC
            cnta_ref[pl.ds(cr, 1), :] = cnta_ref[pl.ds(cr, 1), :] + onehot

        return carry

    jax.lax.fori_loop(0, nvalid, body, 0)

    @pl.when(ch == NCH - 1)
    def _flush():
        c1 = pltpu.make_async_copy(
            acc_ref, sums_ref.at[p, pl.ds(q * QH, QH), :], sem1)
        c1.start()
        c2 = pltpu.make_async_copy(
            cnta_ref, cnt_ref.at[p, pl.ds(q * QC, QC), :], sem2)
        c2.start()
        c1.wait()
        c2.wait()


def _combine_body(tidx_ref, ss_ref, ts_ref, sc_ref, tc_ref, wt_ref, bt_ref,
                  out_ref):
    ti = tidx_ref[0, 0]
    rows = jax.lax.broadcasted_iota(jnp.int32, (TE, 1), 0)
    tf = jnp.sum(jnp.where(rows == ti, wt_ref[...], 0.0), axis=0,
                 keepdims=True) + bt_ref[...]
    sc = sc_ref[0]
    tc = tc_ref[0]
    diff = (ts_ref[0] / jnp.maximum(tc, 1.0)
            - ss_ref[0] / jnp.maximum(sc, 1.0))
    occ = ((sc + tc) > 0.0).astype(jnp.float32)
    out_ref[0] = (diff + tf) * occ


def kernel(pc0s, pc1s, W_feat, b_feat, W_time, b_time, time_idx=0):
    pcs = jnp.concatenate([pc0s, pc1s], axis=0)
    vx = jnp.clip(jnp.floor((pcs[..., 0] - XMIN) / VS), 0, GRID - 1)
    vy = jnp.clip(jnp.floor((pcs[..., 1] - XMIN) / VS), 0, GRID - 1)
    seg = (vx.astype(jnp.int32) * GRID + vy.astype(jnp.int32))

    pcs_p = jnp.pad(pcs, ((0, 0), (0, NPAD - N), (0, 0)))
    seg_p = jnp.pad(seg, ((0, 0), (0, NPAD - N)))
    pc_blocks = pcs_p.reshape(NP4 * NCH, CHUNK, 3)
    seg_blocks = seg_p.reshape(NP4 * NCH, 1, CHUNK)

    sums, cnt = pl.pallas_call(
        _accum_body,
        grid=(NP4, NQ, NCH),
        in_specs=[
            pl.BlockSpec((1, 1, CHUNK),
                         lambda p, q, ch: (p * NCH + ch, 0, 0),
                         memory_space=pltpu.SMEM),
            pl.BlockSpec((1, CHUNK, 3),
                         lambda p, q, ch: (p * NCH + ch, 0, 0)),
            pl.BlockSpec((3, FEAT), lambda p, q, ch: (0, 0)),
            pl.BlockSpec((1, FEAT), lambda p, q, ch: (0, 0)),
        ],
        out_specs=[
            pl.BlockSpec(memory_space=pl.ANY),
            pl.BlockSpec(memory_space=pl.ANY),
        ],
        out_shape=[
            jax.ShapeDtypeStruct((NP4, HALF, 128), jnp.float32),
            jax.ShapeDtypeStruct((NP4, CNT_ROWS, 128), jnp.float32),
        ],
        scratch_shapes=[
            pltpu.VMEM((QH, 128), jnp.float32),
            pltpu.VMEM((QC, 128), jnp.float32),
            pltpu.VMEM((CHUNK, FEAT), jnp.float32),
            pltpu.SemaphoreType.DMA,
            pltpu.SemaphoreType.DMA,
        ],
    )(seg_blocks, pc_blocks, W_feat, b_feat.reshape(1, FEAT))

    sums4 = sums.reshape(NP4, GG, FEAT)
    cnt4 = cnt.reshape(NP4, GG, 1)
    tidx = jnp.asarray(time_idx, jnp.int32).reshape(1, 1)

    T = 2048
    NT = GG // T
    out = pl.pallas_call(
        _combine_body,
        grid=(B, NT),
        in_specs=[
            pl.BlockSpec(memory_space=pltpu.SMEM),
            pl.BlockSpec((1, T, FEAT), lambda b, t: (b, t, 0)),
            pl.BlockSpec((1, T, FEAT), lambda b, t: (B + b, t, 0)),
            pl.BlockSpec((1, T, 1), lambda b, t: (b, t, 0)),
            pl.BlockSpec((1, T, 1), lambda b, t: (B + b, t, 0)),
            pl.BlockSpec((TE, FEAT), lambda b, t: (0, 0)),
            pl.BlockSpec((1, FEAT), lambda b, t: (0, 0)),
        ],
        out_specs=pl.BlockSpec((1, T, FEAT), lambda b, t: (b, t, 0)),
        out_shape=jax.ShapeDtypeStruct((B, GG, FEAT), jnp.float32),
    )(tidx, sums4, sums4, cnt4, cnt4, W_time, b_time.reshape(1, FEAT))

    return out.reshape(B, GRID, GRID, FEAT)


# speedup vs baseline: 1.0102x; 1.0102x over previous
import jax
import jax.numpy as jnp
from jax.experimental import pallas as pl
from jax.experimental.pallas import tpu as pltpu

B = 2
N = 100000
FEAT = 64
GRID = 512
TE = 4
XMIN = -51.2
VS = 0.2

GG = GRID * GRID
HALF = GG // 2
CHUNK = 2048
NCH = (N + CHUNK - 1) // CHUNK
NPAD = NCH * CHUNK
NP4 = 2 * B
CNT_ROWS = GG // 128
NQ = 2
QH = HALF // NQ
QC = CNT_ROWS // NQ


def _accum_body(seg_ref, pc_ref, wf_ref, bf_ref, sums_ref, cnt_ref,
                acc_ref, cnta_ref, feats_ref, sem1, sem2):
    p = pl.program_id(0)
    q = pl.program_id(1)
    ch = pl.program_id(2)

    @pl.when(ch == 0)
    def _zero():
        acc_ref[...] = jnp.zeros(acc_ref.shape, jnp.float32)
        cnta_ref[...] = jnp.zeros(cnta_ref.shape, jnp.float32)

    feats_ref[...] = jax.nn.relu(pc_ref[0] @ wf_ref[...] + bf_ref[...])

    lane = jax.lax.broadcasted_iota(jnp.int32, (1, 128), 1)
    zero64 = jnp.zeros((1, FEAT), jnp.float32)
    nvalid = jnp.minimum(CHUNK, N - ch * CHUNK)
    seg_lo = q * (GG // NQ)
    seg_hi = seg_lo + GG // NQ

    def body(i8, carry):
        for k in range(8):
            i = i8 * 8 + k
            s = seg_ref[0, 0, i]

            @pl.when(jnp.logical_and(s >= seg_lo, s < seg_hi))
            def _upd():
                f = feats_ref[pl.ds(i, 1), :]
                placed = jnp.where(s % 2 == 0,
                                   jnp.concatenate([f, zero64], axis=1),
                                   jnp.concatenate([zero64, f], axis=1))
                r = s // 2 - q * QH
                acc_ref[pl.ds(r, 1), :] = acc_ref[pl.ds(r, 1), :] + placed
                onehot = (lane == s % 128).astype(jnp.float32)
                cr = s // 128 - q * QC
                cnta_ref[pl.ds(cr, 1), :] = (cnta_ref[pl.ds(cr, 1), :]
                                             + onehot)

        return carry

    jax.lax.fori_loop(0, nvalid // 8, body, 0)

    @pl.when(ch == NCH - 1)
    def _flush():
        c1 = pltpu.make_async_copy(
            acc_ref, sums_ref.at[p, pl.ds(q * QH, QH), :], sem1)
        c1.start()
        c2 = pltpu.make_async_copy(
            cnta_ref, cnt_ref.at[p, pl.ds(q * QC, QC), :], sem2)
        c2.start()
        c1.wait()
        c2.wait()


def _combine_body(tidx_ref, ss_ref, ts_ref, sc_ref, tc_ref, wt_ref, bt_ref,
                  out_ref):
    ti = tidx_ref[0, 0]
    rows = jax.lax.broadcasted_iota(jnp.int32, (TE, 1), 0)
    tf = jnp.sum(jnp.where(rows == ti, wt_ref[...], 0.0), axis=0,
                 keepdims=True) + bt_ref[...]
    sc = sc_ref[0]
    tc = tc_ref[0]
    diff = (ts_ref[0] / jnp.maximum(tc, 1.0)
            - ss_ref[0] / jnp.maximum(sc, 1.0))
    occ = ((sc + tc) > 0.0).astype(jnp.float32)
    out_ref[0] = (diff + tf) * occ


def kernel(pc0s, pc1s, W_feat, b_feat, W_time, b_time, time_idx=0):
    pcs = jnp.concatenate([pc0s, pc1s], axis=0)
    vx = jnp.clip(jnp.floor((pcs[..., 0] - XMIN) / VS), 0, GRID - 1)
    vy = jnp.clip(jnp.floor((pcs[..., 1] - XMIN) / VS), 0, GRID - 1)
    seg = (vx.astype(jnp.int32) * GRID + vy.astype(jnp.int32))

    pcs_p = jnp.pad(pcs, ((0, 0), (0, NPAD - N), (0, 0)))
    seg_p = jnp.pad(seg, ((0, 0), (0, NPAD - N)))
    pc_blocks = pcs_p.reshape(NP4 * NCH, CHUNK, 3)
    seg_blocks = seg_p.reshape(NP4 * NCH, 1, CHUNK)

    sums, cnt = pl.pallas_call(
        _accum_body,
        grid=(NP4, NQ, NCH),
        in_specs=[
            pl.BlockSpec((1, 1, CHUNK),
                         lambda p, q, ch: (p * NCH + ch, 0, 0),
                         memory_space=pltpu.SMEM),
            pl.BlockSpec((1, CHUNK, 3),
                         lambda p, q, ch: (p * NCH + ch, 0, 0)),
            pl.BlockSpec((3, FEAT), lambda p, q, ch: (0, 0)),
            pl.BlockSpec((1, FEAT), lambda p, q, ch: (0, 0)),
        ],
        out_specs=[
            pl.BlockSpec(memory_space=pl.ANY),
            pl.BlockSpec(memory_space=pl.ANY),
        ],
        out_shape=[
            jax.ShapeDtypeStruct((NP4, HALF, 128), jnp.float32),
            jax.ShapeDtypeStruct((NP4, CNT_ROWS, 128), jnp.float32),
        ],
        scratch_shapes=[
            pltpu.VMEM((QH, 128), jnp.float32),
            pltpu.VMEM((QC, 128), jnp.float32),
            pltpu.VMEM((CHUNK, FEAT), jnp.float32),
            pltpu.SemaphoreType.DMA,
            pltpu.SemaphoreType.DMA,
        ],
    )(seg_blocks, pc_blocks, W_feat, b_feat.reshape(1, FEAT))

    sums4 = sums.reshape(NP4, GG, FEAT)
    cnt4 = cnt.reshape(NP4, GG, 1)
    tidx = jnp.asarray(time_idx, jnp.int32).reshape(1, 1)

    T = 2048
    NT = GG // T
    out = pl.pallas_call(
        _combine_body,
        grid=(B, NT),
        in_specs=[
            pl.BlockSpec(memory_space=pltpu.SMEM),
            pl.BlockSpec((1, T, FEAT), lambda b, t: (b, t, 0)),
            pl.BlockSpec((1, T, FEAT), lambda b, t: (B + b, t, 0)),
            pl.BlockSpec((1, T, 1), lambda b, t: (b, t, 0)),
            pl.BlockSpec((1, T, 1), lambda b, t: (B + b, t, 0)),
            pl.BlockSpec((TE, FEAT), lambda b, t: (0, 0)),
            pl.BlockSpec((1, FEAT), lambda b, t: (0, 0)),
        ],
        out_specs=pl.BlockSpec((1, T, FEAT), lambda b, t: (b, t, 0)),
        out_shape=jax.ShapeDtypeStruct((B, GG, FEAT), jnp.float32),
    )(tidx, sums4, sums4, cnt4, cnt4, W_time, b_time.reshape(1, FEAT))

    return out.reshape(B, GRID, GRID, FEAT)
